# Initial kernel scaffold; baseline (speedup 1.0000x reference)
import jax
import jax.numpy as jnp
from jax.experimental import pallas as pl


def kernel(feature, adj, shuf, sparse, msk, samp_bias1, samp_bias2, W_gcn, W_disc, b_disc, W_discAll, b_discAll, H):
    raise NotImplementedError("write your pallas kernel here")


# speedup vs baseline: 1.3030x; 1.3030x over previous
import jax
import jax.numpy as jnp
from jax.experimental import pallas as pl

_HID = 64


def _xw_kernel(f_ref, s_ref, w_ref, o_ref):
    w = w_ref[0]
    a = jnp.dot(f_ref[0], w, preferred_element_type=jnp.float32)
    b = jnp.dot(s_ref[0], w, preferred_element_type=jnp.float32)
    o_ref[0] = jnp.concatenate([a, b], axis=-1)


def _prop_kernel(a_ref, xw_ref, o_ref):
    o_ref[0] = jnp.maximum(
        jnp.dot(a_ref[0], xw_ref[0], preferred_element_type=jnp.float32), 0.0)


def _epi_kernel(h_ref, wd_ref, wda_ref, bd_ref, bda_ref, s1_ref, s2_ref,
                hr_ref, l0_ref, l1_ref, l2_ref, reg_ref):
    s1 = s1_ref[...]
    s2 = s2_ref[...]

    def scores(h1, h2, w, b, l_ref):
        c = jax.nn.sigmoid(jnp.mean(h1, axis=0, keepdims=True))
        wc = jnp.dot(w, c.T, preferred_element_type=jnp.float32)
        p = jnp.dot(h1, wc, preferred_element_type=jnp.float32)
        m = jnp.dot(h2, wc, preferred_element_type=jnp.float32)
        l_ref[0:1, :] = p.reshape(1, -1) + b[0, 0] + s1
        l_ref[1:2, :] = m.reshape(1, -1) + b[0, 0] + s2

    h1_0 = h_ref[0, :, :_HID]
    h2_0 = h_ref[0, :, _HID:]
    h1_1 = h_ref[1, :, :_HID]
    h2_1 = h_ref[1, :, _HID:]
    scores(h1_0, h2_0, wd_ref[...], bd_ref[...], l0_ref)
    scores(h1_1, h2_1, wd_ref[...], bd_ref[...], l1_ref)
    h1a = (h1_0 + h1_1) * 0.5
    h2a = (h2_0 + h2_1) * 0.5
    scores(h1a, h2a, wda_ref[...], bda_ref[...], l2_ref)
    hr = hr_ref[0]
    d1 = hr - h1a
    d2 = hr - h2a
    reg_ref[...] = (jnp.sum(d1 * d1) - jnp.sum(d2 * d2)).reshape(1, 1)


def kernel(feature, adj, shuf, sparse, msk, samp_bias1, samp_bias2,
           W_gcn, W_disc, b_disc, W_discAll, b_discAll, H):
    G, _, N, FT = feature.shape
    hid = W_gcn.shape[-1]
    f = feature.reshape(G, N, FT)
    s = shuf.reshape(G, N, FT)
    a = adj.reshape(G, N, N)

    bn = 2048
    xw = pl.pallas_call(
        _xw_kernel,
        grid=(G, N // bn),
        in_specs=[
            pl.BlockSpec((1, bn, FT), lambda g, i: (g, i, 0)),
            pl.BlockSpec((1, bn, FT), lambda g, i: (g, i, 0)),
            pl.BlockSpec((1, FT, hid), lambda g, i: (g, 0, 0)),
        ],
        out_specs=pl.BlockSpec((1, bn, 2 * hid), lambda g, i: (g, i, 0)),
        out_shape=jax.ShapeDtypeStruct((G, N, 2 * hid), jnp.float32),
    )(f, s, W_gcn)

    bm = 256
    h = pl.pallas_call(
        _prop_kernel,
        grid=(G, N // bm),
        in_specs=[
            pl.BlockSpec((1, bm, N), lambda g, i: (g, i, 0)),
            pl.BlockSpec((1, N, 2 * hid), lambda g, i: (g, 0, 0)),
        ],
        out_specs=pl.BlockSpec((1, bm, 2 * hid), lambda g, i: (g, i, 0)),
        out_shape=jax.ShapeDtypeStruct((G, N, 2 * hid), jnp.float32),
    )(a, xw)

    l0, l1, l2, reg = pl.pallas_call(
        _epi_kernel,
        out_shape=[
            jax.ShapeDtypeStruct((2, N), jnp.float32),
            jax.ShapeDtypeStruct((2, N), jnp.float32),
            jax.ShapeDtypeStruct((2, N), jnp.float32),
            jax.ShapeDtypeStruct((1, 1), jnp.float32),
        ],
    )(h, W_disc, W_discAll, b_disc.reshape(1, 1), b_discAll.reshape(1, 1),
      samp_bias1, samp_bias2, H)

    return (l0.reshape(1, 2 * N), l1.reshape(1, 2 * N),
            l2.reshape(1, 2 * N), reg.reshape(()))


# speedup vs baseline: 1.3183x; 1.0117x over previous
import jax
import jax.numpy as jnp
from jax.experimental import pallas as pl

_HID = 64


def _xw_kernel(f_ref, s_ref, w_ref, o_ref):
    w = w_ref[0]
    a = jnp.dot(f_ref[0], w, preferred_element_type=jnp.float32)
    b = jnp.dot(s_ref[0], w, preferred_element_type=jnp.float32)
    o_ref[0] = jnp.concatenate([a, b], axis=-1).astype(jnp.bfloat16)


def _prop_kernel(a_ref, xw_ref, o_ref):
    o_ref[0] = jnp.maximum(
        jnp.dot(a_ref[0].astype(jnp.bfloat16), xw_ref[0],
                preferred_element_type=jnp.float32), 0.0)


def _epi_kernel(h_ref, wd_ref, wda_ref, bd_ref, bda_ref, s1_ref, s2_ref,
                hr_ref, l0_ref, l1_ref, l2_ref, reg_ref):
    s1 = s1_ref[...]
    s2 = s2_ref[...]

    def scores(h1, h2, w, b, l_ref):
        c = jax.nn.sigmoid(jnp.mean(h1, axis=0, keepdims=True))
        wc = jnp.dot(w, c.T, preferred_element_type=jnp.float32)
        p = jnp.dot(h1, wc, preferred_element_type=jnp.float32)
        m = jnp.dot(h2, wc, preferred_element_type=jnp.float32)
        l_ref[0:1, :] = p.reshape(1, -1) + b[0, 0] + s1
        l_ref[1:2, :] = m.reshape(1, -1) + b[0, 0] + s2

    h1_0 = h_ref[0, :, :_HID]
    h2_0 = h_ref[0, :, _HID:]
    h1_1 = h_ref[1, :, :_HID]
    h2_1 = h_ref[1, :, _HID:]
    scores(h1_0, h2_0, wd_ref[...], bd_ref[...], l0_ref)
    scores(h1_1, h2_1, wd_ref[...], bd_ref[...], l1_ref)
    h1a = (h1_0 + h1_1) * 0.5
    h2a = (h2_0 + h2_1) * 0.5
    scores(h1a, h2a, wda_ref[...], bda_ref[...], l2_ref)
    hr = hr_ref[0]
    d1 = hr - h1a
    d2 = hr - h2a
    reg_ref[...] = (jnp.sum(d1 * d1) - jnp.sum(d2 * d2)).reshape(1, 1)


def kernel(feature, adj, shuf, sparse, msk, samp_bias1, samp_bias2,
           W_gcn, W_disc, b_disc, W_discAll, b_discAll, H):
    G, _, N, FT = feature.shape
    hid = W_gcn.shape[-1]
    f = feature.reshape(G, N, FT)
    s = shuf.reshape(G, N, FT)
    a = adj.reshape(G, N, N)

    bn = 2048
    xw = pl.pallas_call(
        _xw_kernel,
        grid=(G, N // bn),
        in_specs=[
            pl.BlockSpec((1, bn, FT), lambda g, i: (g, i, 0)),
            pl.BlockSpec((1, bn, FT), lambda g, i: (g, i, 0)),
            pl.BlockSpec((1, FT, hid), lambda g, i: (g, 0, 0)),
        ],
        out_specs=pl.BlockSpec((1, bn, 2 * hid), lambda g, i: (g, i, 0)),
        out_shape=jax.ShapeDtypeStruct((G, N, 2 * hid), jnp.bfloat16),
    )(f, s, W_gcn)

    bm = 256
    h = pl.pallas_call(
        _prop_kernel,
        grid=(G, N // bm),
        in_specs=[
            pl.BlockSpec((1, bm, N), lambda g, i: (g, i, 0)),
            pl.BlockSpec((1, N, 2 * hid), lambda g, i: (g, 0, 0)),
        ],
        out_specs=pl.BlockSpec((1, bm, 2 * hid), lambda g, i: (g, i, 0)),
        out_shape=jax.ShapeDtypeStruct((G, N, 2 * hid), jnp.float32),
    )(a, xw)

    l0, l1, l2, reg = pl.pallas_call(
        _epi_kernel,
        out_shape=[
            jax.ShapeDtypeStruct((2, N), jnp.float32),
            jax.ShapeDtypeStruct((2, N), jnp.float32),
            jax.ShapeDtypeStruct((2, N), jnp.float32),
            jax.ShapeDtypeStruct((1, 1), jnp.float32),
        ],
    )(h, W_disc, W_discAll, b_disc.reshape(1, 1), b_discAll.reshape(1, 1),
      samp_bias1, samp_bias2, H)

    return (l0.reshape(1, 2 * N), l1.reshape(1, 2 * N),
            l2.reshape(1, 2 * N), reg.reshape(()))


# speedup vs baseline: 1.4133x; 1.0721x over previous
import jax
import jax.numpy as jnp
from jax.experimental import pallas as pl

_HID = 64


def _xw_kernel(f_ref, s_ref, w_ref, o_ref):
    w = w_ref[0]
    a = jnp.dot(f_ref[0], w, preferred_element_type=jnp.float32)
    b = jnp.dot(s_ref[0], w, preferred_element_type=jnp.float32)
    o_ref[0] = jnp.concatenate([a, b], axis=-1).astype(jnp.bfloat16)


def _prop_kernel(a_ref, xw_ref, o_ref):
    o_ref[0] = jnp.maximum(
        jnp.dot(a_ref[0].astype(jnp.bfloat16), xw_ref[0],
                preferred_element_type=jnp.float32), 0.0)


def _epi_kernel(h_ref, wd_ref, wda_ref, bd_ref, bda_ref, s1_ref, s2_ref,
                hr_ref, sc_ref, reg_ref):
    s1 = s1_ref[...]
    s2 = s2_ref[...]
    wd = wd_ref[...]
    wda = wda_ref[...]
    bd = bd_ref[...]
    bda = bda_ref[...]

    h1_0 = h_ref[0, :, :_HID]
    h2_0 = h_ref[0, :, _HID:]
    h1_1 = h_ref[1, :, :_HID]
    h2_1 = h_ref[1, :, _HID:]

    c0 = jax.nn.sigmoid(jnp.mean(h1_0, axis=0, keepdims=True))
    c1 = jax.nn.sigmoid(jnp.mean(h1_1, axis=0, keepdims=True))
    ca = jax.nn.sigmoid(0.5 * (jnp.mean(h1_0, axis=0, keepdims=True)
                               + jnp.mean(h1_1, axis=0, keepdims=True)))
    wc0 = jnp.dot(wd, c0.T, preferred_element_type=jnp.float32)
    wc1 = jnp.dot(wd, c1.T, preferred_element_type=jnp.float32)
    wca = jnp.dot(wda, ca.T, preferred_element_type=jnp.float32)

    z = jnp.zeros_like(wc0)
    b0 = jnp.concatenate([
        jnp.concatenate([wc0, z, wca, z], axis=1),
        jnp.concatenate([z, wc0, z, wca], axis=1)], axis=0)
    b1 = jnp.concatenate([
        jnp.concatenate([wc1, z, wca, z], axis=1),
        jnp.concatenate([z, wc1, z, wca], axis=1)], axis=0)
    o0 = jnp.dot(h_ref[0], b0, preferred_element_type=jnp.float32)
    o1 = jnp.dot(h_ref[1], b1, preferred_element_type=jnp.float32)

    sc_ref[:, 0:1] = o0[:, 0:1] + bd + s1
    sc_ref[:, 1:2] = o0[:, 1:2] + bd + s2
    sc_ref[:, 2:3] = o1[:, 0:1] + bd + s1
    sc_ref[:, 3:4] = o1[:, 1:2] + bd + s2
    sc_ref[:, 4:5] = 0.5 * (o0[:, 2:3] + o1[:, 2:3]) + bda + s1
    sc_ref[:, 5:6] = 0.5 * (o0[:, 3:4] + o1[:, 3:4]) + bda + s2
    sc_ref[:, 6:8] = jnp.zeros_like(sc_ref[:, 6:8])

    h1a = (h1_0 + h1_1) * 0.5
    h2a = (h2_0 + h2_1) * 0.5
    hr = hr_ref[0]
    d1 = hr - h1a
    d2 = hr - h2a
    reg_ref[...] = (jnp.sum(d1 * d1) - jnp.sum(d2 * d2)).reshape(1, 1)


def kernel(feature, adj, shuf, sparse, msk, samp_bias1, samp_bias2,
           W_gcn, W_disc, b_disc, W_discAll, b_discAll, H):
    G, _, N, FT = feature.shape
    hid = W_gcn.shape[-1]
    f = feature.reshape(G, N, FT)
    s = shuf.reshape(G, N, FT)
    a = adj.reshape(G, N, N)

    bn = 2048
    xw = pl.pallas_call(
        _xw_kernel,
        grid=(G, N // bn),
        in_specs=[
            pl.BlockSpec((1, bn, FT), lambda g, i: (g, i, 0)),
            pl.BlockSpec((1, bn, FT), lambda g, i: (g, i, 0)),
            pl.BlockSpec((1, FT, hid), lambda g, i: (g, 0, 0)),
        ],
        out_specs=pl.BlockSpec((1, bn, 2 * hid), lambda g, i: (g, i, 0)),
        out_shape=jax.ShapeDtypeStruct((G, N, 2 * hid), jnp.bfloat16),
    )(f, s, W_gcn)

    bm = 512
    h = pl.pallas_call(
        _prop_kernel,
        grid=(G, N // bm),
        in_specs=[
            pl.BlockSpec((1, bm, N), lambda g, i: (g, i, 0)),
            pl.BlockSpec((1, N, 2 * hid), lambda g, i: (g, 0, 0)),
        ],
        out_specs=pl.BlockSpec((1, bm, 2 * hid), lambda g, i: (g, i, 0)),
        out_shape=jax.ShapeDtypeStruct((G, N, 2 * hid), jnp.float32),
    )(a, xw)

    sc, reg = pl.pallas_call(
        _epi_kernel,
        out_shape=[
            jax.ShapeDtypeStruct((N, 8), jnp.float32),
            jax.ShapeDtypeStruct((1, 1), jnp.float32),
        ],
    )(h, W_disc, W_discAll, b_disc.reshape(1, 1), b_discAll.reshape(1, 1),
      samp_bias1.reshape(N, 1), samp_bias2.reshape(N, 1), H)

    sct = sc[:, :6].T
    return (sct[0:2].reshape(1, 2 * N), sct[2:4].reshape(1, 2 * N),
            sct[4:6].reshape(1, 2 * N), reg.reshape(()))


# speedup vs baseline: 1.4967x; 1.0590x over previous
import jax
import jax.numpy as jnp
from jax.experimental import pallas as pl
from jax.experimental.pallas import tpu as pltpu


def kernel(feature, adj, shuf, sparse, msk, samp_bias1, samp_bias2,
           W_gcn, W_disc, b_disc, W_discAll, b_discAll, H):
    G, _, N, FT = feature.shape
    hid = W_gcn.shape[-1]
    f = feature.reshape(G, N, FT)
    s = shuf.reshape(G, N, FT)
    a = adj.reshape(G, N, N)
    bm = 512
    ni = N // bm

    def fused(f_ref, sh_ref, a_ref, w_ref, wd_ref, wda_ref, bd_ref, bda_ref,
              s1_ref, s2_ref, hr_ref, sc_ref, reg_ref, xw_s, h_s):
        g = pl.program_id(0)
        i = pl.program_id(1)

        @pl.when(i == 0)
        def _():
            w = w_ref[0]
            p1 = jnp.dot(f_ref[0], w, preferred_element_type=jnp.float32)
            p2 = jnp.dot(sh_ref[0], w, preferred_element_type=jnp.float32)
            xw_s[...] = jnp.concatenate([p1, p2], axis=-1)

        hblk = jnp.maximum(
            jnp.dot(a_ref[0], xw_s[...],
                    preferred_element_type=jnp.float32), 0.0)
        h_s[pl.ds(g * N + i * bm, bm), :] = hblk

        @pl.when(jnp.logical_and(g == G - 1, i == ni - 1))
        def _():
            s1 = s1_ref[...]
            s2 = s2_ref[...]
            wd = wd_ref[...]
            wda = wda_ref[...]
            bd = bd_ref[...]
            bda = bda_ref[...]
            h1_0 = h_s[0:N, 0:hid]
            h2_0 = h_s[0:N, hid:]
            h1_1 = h_s[N:, 0:hid]
            h2_1 = h_s[N:, hid:]

            m0 = jnp.mean(h1_0, axis=0, keepdims=True)
            m1 = jnp.mean(h1_1, axis=0, keepdims=True)
            c0 = jax.nn.sigmoid(m0)
            c1 = jax.nn.sigmoid(m1)
            ca = jax.nn.sigmoid(0.5 * (m0 + m1))
            wc0 = jnp.dot(wd, c0.T, preferred_element_type=jnp.float32)
            wc1 = jnp.dot(wd, c1.T, preferred_element_type=jnp.float32)
            wca = jnp.dot(wda, ca.T, preferred_element_type=jnp.float32)

            z = jnp.zeros_like(wc0)
            b0 = jnp.concatenate([
                jnp.concatenate([wc0, z, wca, z], axis=1),
                jnp.concatenate([z, wc0, z, wca], axis=1)], axis=0)
            b1 = jnp.concatenate([
                jnp.concatenate([wc1, z, wca, z], axis=1),
                jnp.concatenate([z, wc1, z, wca], axis=1)], axis=0)
            o0 = jnp.dot(h_s[0:N, :], b0, preferred_element_type=jnp.float32)
            o1 = jnp.dot(h_s[N:, :], b1, preferred_element_type=jnp.float32)

            sc_ref[:, 0:1] = o0[:, 0:1] + bd + s1
            sc_ref[:, 1:2] = o0[:, 1:2] + bd + s2
            sc_ref[:, 2:3] = o1[:, 0:1] + bd + s1
            sc_ref[:, 3:4] = o1[:, 1:2] + bd + s2
            sc_ref[:, 4:5] = 0.5 * (o0[:, 2:3] + o1[:, 2:3]) + bda + s1
            sc_ref[:, 5:6] = 0.5 * (o0[:, 3:4] + o1[:, 3:4]) + bda + s2
            sc_ref[:, 6:8] = jnp.zeros_like(sc_ref[:, 6:8])

            h1a = (h1_0 + h1_1) * 0.5
            h2a = (h2_0 + h2_1) * 0.5
            hr = hr_ref[0]
            d1 = hr - h1a
            d2 = hr - h2a
            reg_ref[...] = (jnp.sum(d1 * d1) - jnp.sum(d2 * d2)).reshape(1, 1)

    sc, reg = pl.pallas_call(
        fused,
        grid=(G, ni),
        in_specs=[
            pl.BlockSpec((1, N, FT), lambda g, i: (g, 0, 0)),
            pl.BlockSpec((1, N, FT), lambda g, i: (g, 0, 0)),
            pl.BlockSpec((1, bm, N), lambda g, i: (g, i, 0)),
            pl.BlockSpec((1, FT, hid), lambda g, i: (g, 0, 0)),
            pl.BlockSpec((hid, hid), lambda g, i: (0, 0)),
            pl.BlockSpec((hid, hid), lambda g, i: (0, 0)),
            pl.BlockSpec((1, 1), lambda g, i: (0, 0)),
            pl.BlockSpec((1, 1), lambda g, i: (0, 0)),
            pl.BlockSpec((N, 1), lambda g, i: (0, 0)),
            pl.BlockSpec((N, 1), lambda g, i: (0, 0)),
            pl.BlockSpec((1, N, hid), lambda g, i: (0, 0, 0)),
        ],
        out_specs=[
            pl.BlockSpec((N, 8), lambda g, i: (0, 0)),
            pl.BlockSpec((1, 1), lambda g, i: (0, 0)),
        ],
        out_shape=[
            jax.ShapeDtypeStruct((N, 8), jnp.float32),
            jax.ShapeDtypeStruct((1, 1), jnp.float32),
        ],
        scratch_shapes=[
            pltpu.VMEM((N, 2 * hid), jnp.float32),
            pltpu.VMEM((G * N, 2 * hid), jnp.float32),
        ],
    )(f, s, a, W_gcn, W_disc, W_discAll,
      b_disc.reshape(1, 1), b_discAll.reshape(1, 1),
      samp_bias1.reshape(N, 1), samp_bias2.reshape(N, 1), H)

    sct = sc[:, :6].T
    return (sct[0:2].reshape(1, 2 * N), sct[2:4].reshape(1, 2 * N),
            sct[4:6].reshape(1, 2 * N), reg.reshape(()))
